# baseline (speedup 1.0000x reference)
import jax
import jax.numpy as jnp
from jax import lax
from jax.experimental import pallas as pl
from jax.experimental.pallas import tpu as pltpu
from jax.experimental.pallas import tpu_sc as plsc

_NUM_EXPERTS = 256
_N_GROUPS = 8
_TOPK_GROUP = 4
_TOPK_EXPERTS = 8
_SCALE = 2.5
_EPS = 1e-20
_BATCH = 128
_HIDDEN = 4096

_NC = 2
_NS = 16
_NW = _NC * _NS
_TPW = _BATCH // _NW
_NV = _NUM_EXPERTS // 16

_NEG_INF = float("-inf")
_BIG_I32 = 1 << 30


_N_CHUNKS = 2
_CHUNK = _NUM_EXPERTS // _N_CHUNKS


def _scores_tc_kernel(x_ref, w_ref, out_ref):
    j = pl.program_id(0)
    out_ref[:, pl.ds(j * _CHUNK, _CHUNK)] = jnp.dot(
        x_ref[...], w_ref[...], preferred_element_type=jnp.float32)

    @pl.when(j == _N_CHUNKS - 1)
    def _():
        logits = out_ref[...]
        m = jnp.max(logits, axis=-1, keepdims=True)
        e = jnp.exp(logits - m)
        out_ref[...] = e / jnp.sum(e, axis=-1, keepdims=True)


def _bfly(v, iota, op):
    for sh in (8, 4, 2, 1):
        v = op(v, v[iota ^ sh])
    return v


def _one_token(t, sb_v, scores_v, bias_v, pw_v, pi_v, iota):
    sb = []
    for j in range(_NV):
        v = scores_v[t, pl.ds(j * 16, 16)] + bias_v[pl.ds(j * 16, 16)]
        sb_v[pl.ds(j * 16, 16)] = v
        sb.append(v)

    gv = jnp.full((16,), _NEG_INF, jnp.float32)
    for g in range(_N_GROUPS):
        a, c = sb[2 * g], sb[2 * g + 1]
        m = jnp.maximum(a, c)
        s = jnp.minimum(a, c)
        for sh in (8, 4, 2, 1):
            ms = m[iota ^ sh]
            ss = s[iota ^ sh]
            s = jnp.maximum(jnp.minimum(m, ms), jnp.maximum(s, ss))
            m = jnp.maximum(m, ms)
        gv = jnp.where(iota == g, m + s, gv)

    gsel = []
    for _ in range(_TOPK_GROUP):
        mx = _bfly(gv, iota, jnp.maximum)
        gi = _bfly(jnp.where(gv == mx, iota, 16), iota, jnp.minimum)
        gsel.append(gi)
        gv = jnp.where(iota == gi, _NEG_INF, gv)

    cur, ci = [], []
    for p in range(_TOPK_GROUP * 2):
        idx_p = gsel[p // 2] * 32 + (iota + 16 * (p % 2))
        cur.append(plsc.load_gather(sb_v, [idx_p]))
        ci.append(idx_p)

    idxvec = jnp.zeros((16,), jnp.int32)
    for k in range(_TOPK_EXPERTS):
        m8 = cur[0]
        for p in range(1, 8):
            m8 = jnp.maximum(m8, cur[p])
        tmax = _bfly(m8, iota, jnp.maximum)
        mm = jnp.where(cur[0] == tmax, ci[0], _BIG_I32)
        for p in range(1, 8):
            mm = jnp.minimum(mm, jnp.where(cur[p] == tmax, ci[p], _BIG_I32))
        e = _bfly(mm, iota, jnp.minimum)
        idxvec = jnp.where(iota == k, e, idxvec)
        if k < _TOPK_EXPERTS - 1:
            for p in range(8):
                cur[p] = jnp.where(ci[p] == e, _NEG_INF, cur[p])

    tvec = jnp.full((16,), t, jnp.int32)
    wv = plsc.load_gather(scores_v, [tvec, idxvec])
    wv8 = jnp.where(iota < _TOPK_EXPERTS, wv, jnp.float32(0.0))
    denom = _bfly(wv8, iota, jnp.add)
    lanes8 = iota < _TOPK_EXPERTS
    plsc.store_scatter(pw_v, [tvec, iota], (wv8 / denom + _EPS) * _SCALE,
                       mask=lanes8)
    plsc.store_scatter(pi_v, [tvec, iota], idxvec, mask=lanes8)


def _routing_sc_kernel(scores_hbm, bias_hbm, w_hbm, idx_hbm,
                       scores_v, bias_v, sb_v, sb2_v, pw_v, pi_v, dma_sem):
    wid = lax.axis_index("s") * _NC + lax.axis_index("c")
    base = wid * _TPW
    h1 = pltpu.async_copy(scores_hbm.at[pl.ds(base, _TPW)], scores_v, dma_sem)
    h2 = pltpu.async_copy(bias_hbm, bias_v, dma_sem)
    h1.wait()
    h2.wait()
    iota = lax.iota(jnp.int32, 16)

    def tok_pair(p, carry):
        for dt in range(2):
            t = 2 * p + dt
            sbr = sb_v if dt == 0 else sb2_v
            _one_token(t, sbr, scores_v, bias_v, pw_v, pi_v, iota)
        return carry

    lax.fori_loop(0, _TPW // 2, tok_pair, 0)

    h3 = pltpu.async_copy(pw_v, w_hbm.at[0, 0, pl.ds(base, _TPW)], dma_sem)
    h4 = pltpu.async_copy(pi_v, idx_hbm.at[0, 0, pl.ds(base, _TPW)], dma_sem)
    h3.wait()
    h4.wait()


def kernel(x, W, bias):
    x2 = x.reshape(_BATCH, _HIDDEN)
    scores = pl.pallas_call(
        _scores_tc_kernel,
        grid=(_N_CHUNKS,),
        in_specs=[
            pl.BlockSpec((_BATCH, _HIDDEN), lambda j: (0, 0)),
            pl.BlockSpec((_HIDDEN, _CHUNK), lambda j: (0, j)),
        ],
        out_specs=pl.BlockSpec((_BATCH, _NUM_EXPERTS), lambda j: (0, 0)),
        out_shape=jax.ShapeDtypeStruct((_BATCH, _NUM_EXPERTS), jnp.float32),
    )(x2, W)

    routing = pl.kernel(
        _routing_sc_kernel,
        out_type=[
            jax.ShapeDtypeStruct((1, 1, _BATCH, _TOPK_EXPERTS), jnp.float32),
            jax.ShapeDtypeStruct((1, 1, _BATCH, _TOPK_EXPERTS), jnp.int32),
        ],
        mesh=plsc.VectorSubcoreMesh(core_axis_name="c", subcore_axis_name="s"),
        compiler_params=pltpu.CompilerParams(needs_layout_passes=False),
        scratch_types=[
            pltpu.VMEM((_TPW, _NUM_EXPERTS), jnp.float32),
            pltpu.VMEM((_NUM_EXPERTS,), jnp.float32),
            pltpu.VMEM((_NUM_EXPERTS,), jnp.float32),
            pltpu.VMEM((_NUM_EXPERTS,), jnp.float32),
            pltpu.VMEM((_TPW, _TOPK_EXPERTS), jnp.float32),
            pltpu.VMEM((_TPW, _TOPK_EXPERTS), jnp.int32),
            pltpu.SemaphoreType.DMA,
        ],
    )
    w, idx = routing(scores, bias)
    return (w, idx)


# speedup vs baseline: 1.0213x; 1.0213x over previous
import jax
import jax.numpy as jnp
from jax import lax
from jax.experimental import pallas as pl
from jax.experimental.pallas import tpu as pltpu
from jax.experimental.pallas import tpu_sc as plsc

_NUM_EXPERTS = 256
_N_GROUPS = 8
_TOPK_GROUP = 4
_TOPK_EXPERTS = 8
_SCALE = 2.5
_EPS = 1e-20
_BATCH = 128
_HIDDEN = 4096

_NC = 2
_NS = 16
_NW = _NC * _NS
_TPW = _BATCH // _NW
_NV = _NUM_EXPERTS // 16

_NEG_INF = float("-inf")
_BIG_I32 = 1 << 30


_N_CHUNKS = 2
_CHUNK = _NUM_EXPERTS // _N_CHUNKS


def _scores_tc_kernel(x_ref, w_ref, out_ref):
    j = pl.program_id(0)
    out_ref[:, pl.ds(j * _CHUNK, _CHUNK)] = jnp.dot(
        x_ref[...], w_ref[...], preferred_element_type=jnp.float32)

    @pl.when(j == _N_CHUNKS - 1)
    def _():
        logits = out_ref[...]
        m = jnp.max(logits, axis=-1, keepdims=True)
        e = jnp.exp(logits - m)
        out_ref[...] = e / jnp.sum(e, axis=-1, keepdims=True)


def _bfly(v, iota, op):
    for sh in (8, 4, 2, 1):
        v = op(v, v[iota ^ sh])
    return v


def _one_token(t, sb_v, scores_v, bias_v, pw_v, pi_v, iota):
    sb = []
    for j in range(_NV):
        v = scores_v[t, pl.ds(j * 16, 16)] + bias_v[pl.ds(j * 16, 16)]
        sb_v[pl.ds(j * 16, 16)] = v
        sb.append(v)

    gv = jnp.full((16,), _NEG_INF, jnp.float32)
    for g in range(_N_GROUPS):
        a, c = sb[2 * g], sb[2 * g + 1]
        m = jnp.maximum(a, c)
        s = jnp.minimum(a, c)
        for sh in (8, 4, 2, 1):
            ms = m[iota ^ sh]
            ss = s[iota ^ sh]
            s = jnp.maximum(jnp.minimum(m, ms), jnp.maximum(s, ss))
            m = jnp.maximum(m, ms)
        gv = jnp.where(iota == g, m + s, gv)

    gsel = []
    for _ in range(_TOPK_GROUP):
        mx = _bfly(gv, iota, jnp.maximum)
        gi = _bfly(jnp.where(gv == mx, iota, 16), iota, jnp.minimum)
        gsel.append(gi)
        gv = jnp.where(iota == gi, _NEG_INF, gv)

    cur, ci = [], []
    for p in range(_TOPK_GROUP * 2):
        idx_p = gsel[p // 2] * 32 + (iota + 16 * (p % 2))
        cur.append(plsc.load_gather(sb_v, [idx_p]))
        ci.append(idx_p)

    idxvec = jnp.zeros((16,), jnp.int32)
    for k in range(_TOPK_EXPERTS):
        m8 = cur[0]
        for p in range(1, 8):
            m8 = jnp.maximum(m8, cur[p])
        tmax = _bfly(m8, iota, jnp.maximum)
        mm = jnp.where(cur[0] == tmax, ci[0], _BIG_I32)
        for p in range(1, 8):
            mm = jnp.minimum(mm, jnp.where(cur[p] == tmax, ci[p], _BIG_I32))
        e = _bfly(mm, iota, jnp.minimum)
        idxvec = jnp.where(iota == k, e, idxvec)
        if k < _TOPK_EXPERTS - 1:
            for p in range(8):
                cur[p] = jnp.where(ci[p] == e, _NEG_INF, cur[p])

    tvec = jnp.full((16,), t, jnp.int32)
    wv = plsc.load_gather(scores_v, [tvec, idxvec])
    wv8 = jnp.where(iota < _TOPK_EXPERTS, wv, jnp.float32(0.0))
    denom = _bfly(wv8, iota, jnp.add)
    lanes8 = iota < _TOPK_EXPERTS
    plsc.store_scatter(pw_v, [tvec, iota], (wv8 / denom + _EPS) * _SCALE,
                       mask=lanes8)
    plsc.store_scatter(pi_v, [tvec, iota], idxvec, mask=lanes8)


def _routing_sc_kernel(scores_hbm, bias_hbm, w_hbm, idx_hbm,
                       scores_v, bias_v, sb_v, pw_v, pi_v, dma_sem):
    wid = lax.axis_index("s") * _NC + lax.axis_index("c")
    base = wid * _TPW
    h1 = pltpu.async_copy(scores_hbm.at[pl.ds(base, _TPW)], scores_v, dma_sem)
    h2 = pltpu.async_copy(bias_hbm, bias_v, dma_sem)
    h1.wait()
    h2.wait()
    iota = lax.iota(jnp.int32, 16)

    def tok(t, carry):
        _one_token(t, sb_v, scores_v, bias_v, pw_v, pi_v, iota)
        return carry

    lax.fori_loop(0, _TPW, tok, 0)

    h3 = pltpu.async_copy(pw_v, w_hbm.at[0, 0, pl.ds(base, _TPW)], dma_sem)
    h4 = pltpu.async_copy(pi_v, idx_hbm.at[0, 0, pl.ds(base, _TPW)], dma_sem)
    h3.wait()
    h4.wait()


def kernel(x, W, bias):
    x2 = x.reshape(_BATCH, _HIDDEN)
    scores = pl.pallas_call(
        _scores_tc_kernel,
        grid=(_N_CHUNKS,),
        in_specs=[
            pl.BlockSpec((_BATCH, _HIDDEN), lambda j: (0, 0)),
            pl.BlockSpec((_HIDDEN, _CHUNK), lambda j: (0, j)),
        ],
        out_specs=pl.BlockSpec((_BATCH, _NUM_EXPERTS), lambda j: (0, 0)),
        out_shape=jax.ShapeDtypeStruct((_BATCH, _NUM_EXPERTS), jnp.float32),
    )(x2, W)

    routing = pl.kernel(
        _routing_sc_kernel,
        out_type=[
            jax.ShapeDtypeStruct((1, 1, _BATCH, _TOPK_EXPERTS), jnp.float32),
            jax.ShapeDtypeStruct((1, 1, _BATCH, _TOPK_EXPERTS), jnp.int32),
        ],
        mesh=plsc.VectorSubcoreMesh(core_axis_name="c", subcore_axis_name="s"),
        compiler_params=pltpu.CompilerParams(needs_layout_passes=False),
        scratch_types=[
            pltpu.VMEM((_TPW, _NUM_EXPERTS), jnp.float32),
            pltpu.VMEM((_NUM_EXPERTS,), jnp.float32),
            pltpu.VMEM((_NUM_EXPERTS,), jnp.float32),
            pltpu.VMEM((_TPW, _TOPK_EXPERTS), jnp.float32),
            pltpu.VMEM((_TPW, _TOPK_EXPERTS), jnp.int32),
            pltpu.SemaphoreType.DMA,
        ],
    )
    w, idx = routing(scores, bias)
    return (w, idx)


# speedup vs baseline: 1.0266x; 1.0052x over previous
import jax
import jax.numpy as jnp
from jax import lax
from jax.experimental import pallas as pl
from jax.experimental.pallas import tpu as pltpu
from jax.experimental.pallas import tpu_sc as plsc

_NUM_EXPERTS = 256
_N_GROUPS = 8
_TOPK_GROUP = 4
_TOPK_EXPERTS = 8
_SCALE = 2.5
_EPS = 1e-20
_BATCH = 128
_HIDDEN = 4096

_NC = 2
_NS = 16
_NW = _NC * _NS
_TPW = _BATCH // _NW
_NV = _NUM_EXPERTS // 16

_NEG_INF = float("-inf")
_BIG_I32 = 1 << 30


_N_CHUNKS = 2
_CHUNK = _NUM_EXPERTS // _N_CHUNKS


def _scores_tc_kernel(x_ref, w_ref, out_ref):
    j = pl.program_id(0)
    out_ref[:, pl.ds(j * _CHUNK, _CHUNK)] = jnp.dot(
        x_ref[...], w_ref[...], preferred_element_type=jnp.float32)

    @pl.when(j == _N_CHUNKS - 1)
    def _():
        logits = out_ref[...]
        m = jnp.max(logits, axis=-1, keepdims=True)
        e = jnp.exp(logits - m)
        out_ref[...] = e / jnp.sum(e, axis=-1, keepdims=True)


def _bfly(v, iota, op):
    for sh in (8, 4, 2, 1):
        v = op(v, v[iota ^ sh])
    return v


def _one_token(t, sb_v, scores_v, bias_v, pw_v, pi_v, iota):
    sb = []
    for j in range(_NV):
        v = scores_v[t, pl.ds(j * 16, 16)] + bias_v[pl.ds(j * 16, 16)]
        sb_v[pl.ds(j * 16, 16)] = v
        sb.append(v)

    gv = jnp.full((16,), _NEG_INF, jnp.float32)
    for g in range(_N_GROUPS):
        a, c = sb[2 * g], sb[2 * g + 1]
        m = jnp.maximum(a, c)
        s = jnp.minimum(a, c)
        for sh in (8, 4, 2, 1):
            ms = m[iota ^ sh]
            ss = s[iota ^ sh]
            s = jnp.maximum(jnp.minimum(m, ms), jnp.maximum(s, ss))
            m = jnp.maximum(m, ms)
        gv = jnp.where(iota == g, m + s, gv)

    gsel = []
    for _ in range(_TOPK_GROUP):
        mx = _bfly(gv, iota, jnp.maximum)
        gi = _bfly(jnp.where(gv == mx, iota, 16), iota, jnp.minimum)
        gsel.append(gi)
        gv = jnp.where(iota == gi, _NEG_INF, gv)

    cur, ci = [], []
    for p in range(_TOPK_GROUP * 2):
        idx_p = gsel[p // 2] * 32 + (iota + 16 * (p % 2))
        cur.append(plsc.load_gather(sb_v, [idx_p]))
        ci.append(idx_p)

    idxvec = jnp.zeros((16,), jnp.int32)
    for k in range(_TOPK_EXPERTS):
        m8 = cur[0]
        for p in range(1, 8):
            m8 = jnp.maximum(m8, cur[p])
        tmax = _bfly(m8, iota, jnp.maximum)
        mm = jnp.where(cur[0] == tmax, ci[0], _BIG_I32)
        for p in range(1, 8):
            mm = jnp.minimum(mm, jnp.where(cur[p] == tmax, ci[p], _BIG_I32))
        e = _bfly(mm, iota, jnp.minimum)
        idxvec = jnp.where(iota == k, e, idxvec)
        if k < _TOPK_EXPERTS - 1:
            for p in range(8):
                cur[p] = jnp.where(ci[p] == e, _NEG_INF, cur[p])

    tvec = jnp.full((16,), t, jnp.int32)
    wv = plsc.load_gather(scores_v, [tvec, idxvec])
    wv8 = jnp.where(iota < _TOPK_EXPERTS, wv, jnp.float32(0.0))
    denom = _bfly(wv8, iota, jnp.add)
    lanes8 = iota < _TOPK_EXPERTS
    plsc.store_scatter(pw_v, [tvec, iota], (wv8 / denom + _EPS) * _SCALE,
                       mask=lanes8)
    plsc.store_scatter(pi_v, [tvec, iota], idxvec, mask=lanes8)


def _routing_sc_kernel(scores_hbm, bias_hbm, w_hbm, idx_hbm,
                       scores_v, bias_v, sb_v, pw_v, pi_v, dma_sem):
    wid = lax.axis_index("s") * _NC + lax.axis_index("c")
    base = wid * _TPW
    h1 = pltpu.async_copy(scores_hbm.at[pl.ds(base, _TPW)], scores_v, dma_sem)
    h2 = pltpu.async_copy(bias_hbm, bias_v, dma_sem)
    h1.wait()
    h2.wait()
    iota = lax.iota(jnp.int32, 16)

    def tok(t, carry):
        _one_token(t, sb_v, scores_v, bias_v, pw_v, pi_v, iota)
        return carry

    lax.fori_loop(0, _TPW, tok, 0)

    h3 = pltpu.async_copy(pw_v, w_hbm.at[0, 0, pl.ds(base, _TPW)], dma_sem)
    h4 = pltpu.async_copy(pi_v, idx_hbm.at[0, 0, pl.ds(base, _TPW)], dma_sem)
    h3.wait()
    h4.wait()


def kernel(x, W, bias):
    x2 = x.reshape(_BATCH, _HIDDEN)
    scores = pl.pallas_call(
        _scores_tc_kernel,
        grid=(_N_CHUNKS,),
        in_specs=[
            pl.BlockSpec((_BATCH, _HIDDEN), lambda j: (0, 0)),
            pl.BlockSpec((_HIDDEN, _CHUNK), lambda j: (0, j)),
        ],
        out_specs=pl.BlockSpec((_BATCH, _NUM_EXPERTS), lambda j: (0, 0)),
        out_shape=jax.ShapeDtypeStruct((_BATCH, _NUM_EXPERTS), jnp.float32),
    )(x2, W)

    routing = pl.kernel(
        _routing_sc_kernel,
        out_type=[
            jax.ShapeDtypeStruct((1, 1, _BATCH, _TOPK_EXPERTS), jnp.float32),
            jax.ShapeDtypeStruct((1, 1, _BATCH, _TOPK_EXPERTS), jnp.int32),
        ],
        mesh=plsc.VectorSubcoreMesh(core_axis_name="c", subcore_axis_name="s"),
        compiler_params=pltpu.CompilerParams(needs_layout_passes=False,
                                             use_tc_tiling_on_sc=True),
        scratch_types=[
            pltpu.VMEM((_TPW, _NUM_EXPERTS), jnp.float32),
            pltpu.VMEM((_NUM_EXPERTS,), jnp.float32),
            pltpu.VMEM((_NUM_EXPERTS,), jnp.float32),
            pltpu.VMEM((_TPW, _TOPK_EXPERTS), jnp.float32),
            pltpu.VMEM((_TPW, _TOPK_EXPERTS), jnp.int32),
            pltpu.SemaphoreType.DMA,
        ],
    )
    w, idx = routing(scores, bias)
    return (w, idx)
